# baseline (speedup 1.0000x reference)
import functools

import jax
import jax.numpy as jnp
from jax import lax
from jax.experimental import pallas as pl
from jax.experimental.pallas import tpu as pltpu
from jax.experimental.pallas import tpu_sc as plsc

N = 10000
E = 320000
D = 128
H = 4
DH = D // H

ROW_BLK = 1000

NC = 2
NS = 16
NW = NC * NS
NP = 10112
RPT = NP // NS
EN = E + N
C = 64
C2 = 2 * C
K = 168
SB = 8
EN_PAD = NW * K * C
NCHUNK = EN_PAD // C


def _proj_body(x_ref, wl_ref, wr_ref, xl_ref, xr_ref):
    x = x_ref[...]
    xl_ref[...] = jnp.dot(x, wl_ref[...], preferred_element_type=jnp.float32)
    xr_ref[...] = jnp.dot(x, wr_ref[...], preferred_element_type=jnp.float32)


@jax.jit
def _proj(x, W_l, W_r):
    grid = (N // ROW_BLK,)
    return pl.pallas_call(
        _proj_body,
        grid=grid,
        in_specs=[
            pl.BlockSpec((ROW_BLK, D), lambda i: (i, 0)),
            pl.BlockSpec((D, D), lambda i: (0, 0)),
            pl.BlockSpec((D, D), lambda i: (0, 0)),
        ],
        out_specs=[
            pl.BlockSpec((ROW_BLK, D), lambda i: (i, 0)),
            pl.BlockSpec((ROW_BLK, D), lambda i: (i, 0)),
        ],
        out_shape=[
            jax.ShapeDtypeStruct((N, D), jnp.float32),
            jax.ShapeDtypeStruct((N, D), jnp.float32),
        ],
    )(x, W_l, W_r)


def _edge_body(xlr_hbm, gm_hbm, dm_hbm, att_hbm, znum_hbm, zden_hbm,
               onum_hbm, oden_hbm,
               acc_num, acc_den,
               buf0, buf1, den0, den1,
               gsb0, gsb1, dsb0, dsb1, att_v,
               gsem0, gsem1, ssem0, ssem1):
    c = lax.axis_index("c")
    s = lax.axis_index("s")
    wid = c * NS + s
    lo = pl.multiple_of(s * RPT, 8)
    row0 = wid * K

    buf_b = (buf0, buf1)
    den_b = (den0, den1)
    gsem = (gsem0, gsem1)
    ssem = (ssem0, ssem1)

    pltpu.sync_copy(znum_hbm.at[pl.ds(lo, RPT)], acc_num.at[pl.ds(lo, RPT)])
    pltpu.sync_copy(zden_hbm.at[pl.ds(lo, RPT)], acc_den.at[pl.ds(lo, RPT)])
    pltpu.sync_copy(att_hbm, att_v)

    zero16 = jnp.zeros((16,), jnp.float32)
    plsc.subcore_barrier()

    lane = lax.iota(jnp.int32, 16)
    xor_idx = [lane ^ 1, lane ^ 2, lane ^ 4, lane ^ 8]
    lane_eq = [lane == h for h in range(H)]
    att_r = [att_v[pl.ds(16 * j, 16)] for j in range(D // 16)]

    def bcast_sum(u):
        dnums = lax.GatherDimensionNumbers(
            offset_dims=(), collapsed_slice_dims=(0,), start_index_map=(0,))
        for xi in xor_idx:
            g = lax.gather(u, xi[:, None], dimension_numbers=dnums,
                           slice_sizes=(1,),
                           mode=lax.GatherScatterMode.PROMISE_IN_BOUNDS)
            u = u + g
        return u

    def load_sb(b):
        q = b & 1

        @pl.when(q == 0)
        def _():
            pltpu.sync_copy(gm_hbm.at[pl.ds(row0 + b * SB, SB)], gsb0)
            pltpu.sync_copy(dm_hbm.at[pl.ds(row0 + b * SB, SB)], dsb0)

        @pl.when(q == 1)
        def _():
            pltpu.sync_copy(gm_hbm.at[pl.ds(row0 + b * SB, SB)], gsb1)
            pltpu.sync_copy(dm_hbm.at[pl.ds(row0 + b * SB, SB)], dsb1)

    def issue_gather(k, p):
        b = k // SB
        r = k - b * SB
        q = b & 1

        GS = C2 // 4

        @pl.when(q == 0)
        def _():
            for t in range(4):
                pltpu.async_copy(xlr_hbm.at[gsb0.at[r, pl.ds(t * GS, GS)]],
                                 buf_b[p].at[pl.ds(t * GS, GS)], gsem[p])

        @pl.when(q == 1)
        def _():
            for t in range(4):
                pltpu.async_copy(xlr_hbm.at[gsb1.at[r, pl.ds(t * GS, GS)]],
                                 buf_b[p].at[pl.ds(t * GS, GS)], gsem[p])

    def wait_gather(p):
        pltpu.make_async_copy(xlr_hbm.at[pl.ds(0, C2)], buf_b[p],
                              gsem[p]).wait()

    def issue_scatter(k, p):
        pass

    def wait_scatter(p):
        pass

    def compute(p):
        buf = buf_b[p]
        den_v = den_b[p]

        def one_edge(e):
            xl = [buf[e, pl.ds(16 * j, 16)] for j in range(D // 16)]
            t = []
            for j in range(D // 16):
                v = xl[j] + buf[C + e, pl.ds(16 * j, 16)]
                lr = jnp.maximum(v, 0.2 * v)
                t.append(lr * att_r[j])
            svecs = []
            for h in range(H):
                u = bcast_sum(t[2 * h] + t[2 * h + 1])
                svecs.append(jnp.exp(u))
            dval = zero16
            for h in range(H):
                dval = jnp.where(lane_eq[h], svecs[h], dval)
            den_v[e, :] = dval
            for j in range(D // 16):
                buf[e, pl.ds(16 * j, 16)] = xl[j] * svecs[j // 2]

        def edge_body(i, carry):
            one_edge(2 * i)
            one_edge(2 * i + 1)
            return carry

        lax.fori_loop(0, C // 2, edge_body, 0)

    load_sb(0)
    issue_gather(0, 0)

    def pair_body(i, carry):
        k0 = 2 * i
        k1 = k0 + 1

        @pl.when(i > 0)
        def _():
            wait_scatter(1)

        issue_gather(k1, 1)
        wait_gather(0)
        compute(0)
        issue_scatter(k0, 0)

        @pl.when(jnp.logical_and((k1 + 1) % SB == 0, k1 + 1 < K))
        def _():
            load_sb((k1 + 1) // SB)

        wait_gather(1)
        compute(1)

        @pl.when(i > 0)
        def _():
            wait_scatter(0)

        @pl.when(k1 + 1 < K)
        def _():
            issue_gather(k1 + 1, 0)

        issue_scatter(k1, 1)
        return carry

    lax.fori_loop(0, K // 2, pair_body, 0)
    wait_scatter(0)
    wait_scatter(1)
    plsc.subcore_barrier()

    pltpu.sync_copy(acc_num.at[pl.ds(lo, RPT)], onum_hbm.at[c, pl.ds(lo, RPT)])
    pltpu.sync_copy(acc_den.at[pl.ds(lo, RPT)], oden_hbm.at[c, pl.ds(lo, RPT)])


@jax.jit
def _edge_sc(xlr, gm, dm, att):
    znum = jnp.zeros((NP, D), jnp.float32)
    zden = jnp.zeros((NP, 16), jnp.float32)
    mesh = plsc.VectorSubcoreMesh(core_axis_name="c", subcore_axis_name="s")
    f = pl.kernel(
        _edge_body,
        out_type=[
            jax.ShapeDtypeStruct((NC, NP, D), jnp.float32),
            jax.ShapeDtypeStruct((NC, NP, 16), jnp.float32),
        ],
        mesh=mesh,
        scratch_types=[
            pltpu.VMEM_SHARED((NP, D), jnp.float32),
            pltpu.VMEM_SHARED((NP, 16), jnp.float32),
            pltpu.VMEM((C2, D), jnp.float32),
            pltpu.VMEM((C2, D), jnp.float32),
            pltpu.VMEM((C, 16), jnp.float32),
            pltpu.VMEM((C, 16), jnp.float32),
            pltpu.VMEM((SB, C2), jnp.int32),
            pltpu.VMEM((SB, C2), jnp.int32),
            pltpu.VMEM((SB, C), jnp.int32),
            pltpu.VMEM((SB, C), jnp.int32),
            pltpu.VMEM((D + 32,), jnp.float32),
            pltpu.SemaphoreType.DMA,
            pltpu.SemaphoreType.DMA,
            pltpu.SemaphoreType.DMA,
            pltpu.SemaphoreType.DMA,
        ],
        compiler_params=pltpu.CompilerParams(needs_layout_passes=False,
                                             use_tc_tiling_on_sc=False),
    )
    return f(xlr, gm, dm, att, znum, zden)


def _post_body(num_ref, den_ref, bias_ref, w1_ref, b1_ref, w2_ref, b2_ref,
               g_ref, bt_ref, y_ref):
    num = num_ref[0] + num_ref[1]
    den = den_ref[0, :, :H] + den_ref[1, :, :H]
    den_full = jnp.repeat(den, DH, axis=1)
    h = num / (den_full + 1e-16) + bias_ref[...]
    t = jnp.maximum(jnp.dot(h, w1_ref[...], preferred_element_type=jnp.float32)
                    + b1_ref[...], 0.0)
    y = jnp.dot(t, w2_ref[...], preferred_element_type=jnp.float32) + b2_ref[...] + h
    mean = jnp.mean(y, axis=-1, keepdims=True)
    yc = y - mean
    var = jnp.mean(yc * yc, axis=-1, keepdims=True)
    y_ref[...] = yc * jax.lax.rsqrt(var + 1e-6) * g_ref[...] + bt_ref[...]


@jax.jit
def _post(onum, oden, bias, W1, b1, W2, b2, gamma, beta):
    grid = (N // ROW_BLK,)
    row3 = lambda i: (0, i, 0)
    fixed = lambda i: (0, 0)
    y = pl.pallas_call(
        _post_body,
        grid=grid,
        in_specs=[
            pl.BlockSpec((NC, ROW_BLK, D), row3),
            pl.BlockSpec((NC, ROW_BLK, 16), row3),
            pl.BlockSpec((1, D), fixed),
            pl.BlockSpec((D, D), fixed),
            pl.BlockSpec((1, D), fixed),
            pl.BlockSpec((D, D), fixed),
            pl.BlockSpec((1, D), fixed),
            pl.BlockSpec((1, D), fixed),
            pl.BlockSpec((1, D), fixed),
        ],
        out_specs=pl.BlockSpec((ROW_BLK, D), lambda i: (i, 0)),
        out_shape=jax.ShapeDtypeStruct((N, D), jnp.float32),
    )(onum, oden, bias.reshape(1, D), W1, b1.reshape(1, D), W2,
      b2.reshape(1, D), gamma.reshape(1, D), beta.reshape(1, D))
    return y[None, :, :]


def kernel(x, edge_index, W_l, W_r, att, bias, W1, b1, W2, b2, gamma, beta):
    xl, xr = _proj(x, W_l, W_r)
    pad_rows = jnp.zeros((NP - N, D), jnp.float32)
    xlr = jnp.concatenate([xl, pad_rows, xr, pad_rows])
    loop = jnp.arange(N, dtype=jnp.int32)
    pad_idx = jnp.full((EN_PAD - EN,), N, jnp.int32)
    src = jnp.concatenate([edge_index[0].astype(jnp.int32), loop, pad_idx])
    dst = jnp.concatenate([edge_index[1].astype(jnp.int32), loop, pad_idx])
    gm = jnp.concatenate(
        [src.reshape(NCHUNK, C), dst.reshape(NCHUNK, C) + NP], axis=1)
    dm = dst.reshape(NCHUNK, C)
    att_flat = jnp.concatenate([att.reshape(D), jnp.zeros((32,), jnp.float32)])
    onum, oden = _edge_sc(xlr, gm, dm, att_flat)
    return _post(onum, oden, bias, W1, b1, W2, b2, gamma, beta)


# speedup vs baseline: 1.0720x; 1.0720x over previous
import functools

import jax
import jax.numpy as jnp
from jax import lax
from jax.experimental import pallas as pl
from jax.experimental.pallas import tpu as pltpu
from jax.experimental.pallas import tpu_sc as plsc

N = 10000
E = 320000
D = 128
H = 4
DH = D // H

ROW_BLK = 1000

NC = 2
NS = 16
NW = NC * NS
NP = 10112
RPT = NP // NS
EN = E + N
C = 64
K = 168
SB = 8
EN_PAD = NW * K * C
NCHUNK = EN_PAD // C


def _proj_body(x_ref, wl_ref, wr_ref, xl_ref, xr_ref):
    x = x_ref[...]
    xl_ref[...] = jnp.dot(x, wl_ref[...], preferred_element_type=jnp.float32)
    xr_ref[...] = jnp.dot(x, wr_ref[...], preferred_element_type=jnp.float32)


@jax.jit
def _proj(x, W_l, W_r):
    grid = (N // ROW_BLK,)
    return pl.pallas_call(
        _proj_body,
        grid=grid,
        in_specs=[
            pl.BlockSpec((ROW_BLK, D), lambda i: (i, 0)),
            pl.BlockSpec((D, D), lambda i: (0, 0)),
            pl.BlockSpec((D, D), lambda i: (0, 0)),
        ],
        out_specs=[
            pl.BlockSpec((ROW_BLK, D), lambda i: (i, 0)),
            pl.BlockSpec((ROW_BLK, D), lambda i: (i, 0)),
        ],
        out_shape=[
            jax.ShapeDtypeStruct((N, D), jnp.float32),
            jax.ShapeDtypeStruct((N, D), jnp.float32),
        ],
    )(x, W_l, W_r)


def _edge_body(xl_hbm, xr_hbm, sm_hbm, dm_hbm, dm64_hbm, att_hbm,
               znum_hbm, zden_hbm,
               onum_hbm, oden_hbm,
               acc_num, acc_den,
               xl0, xl1, xr0, xr1, den0, den1,
               ssb0, ssb1, dgb0, dgb1, dsb0, dsb1, att_v,
               gsem0, gsem1, ssem0, ssem1):
    c = lax.axis_index("c")
    s = lax.axis_index("s")
    wid = c * NS + s
    lo = pl.multiple_of(s * RPT, 8)
    row0 = wid * K

    xl_b = (xl0, xl1)
    xr_b = (xr0, xr1)
    den_b = (den0, den1)
    gsem = (gsem0, gsem1)
    ssem = (ssem0, ssem1)

    pltpu.sync_copy(znum_hbm.at[pl.ds(lo, RPT)], acc_num.at[pl.ds(lo, RPT)])
    pltpu.sync_copy(zden_hbm.at[pl.ds(lo, RPT)], acc_den.at[pl.ds(lo, RPT)])
    pltpu.sync_copy(att_hbm, att_v)

    zero16 = jnp.zeros((16,), jnp.float32)
    plsc.subcore_barrier()

    lane = lax.iota(jnp.int32, 16)
    xor_idx = [lane ^ 1, lane ^ 2, lane ^ 4, lane ^ 8]
    lane_eq = [lane == h for h in range(H)]
    att_r = [att_v[pl.ds(16 * j, 16)] for j in range(D // 16)]

    def bcast_sum(u):
        dnums = lax.GatherDimensionNumbers(
            offset_dims=(), collapsed_slice_dims=(0,), start_index_map=(0,))
        for xi in xor_idx:
            g = lax.gather(u, xi[:, None], dimension_numbers=dnums,
                           slice_sizes=(1,),
                           mode=lax.GatherScatterMode.PROMISE_IN_BOUNDS)
            u = u + g
        return u

    def load_sb(b):
        q = b & 1

        @pl.when(q == 0)
        def _():
            pltpu.sync_copy(sm_hbm.at[pl.ds(2 * (row0 + b * SB), 2 * SB)],
                            ssb0)
            for t in range(2):
                pltpu.sync_copy(dm_hbm.at[t, pl.ds(row0 + b * SB, SB)],
                                dgb0.at[t])
            pltpu.sync_copy(dm64_hbm.at[pl.ds(row0 + b * SB, SB)], dsb0)

        @pl.when(q == 1)
        def _():
            pltpu.sync_copy(sm_hbm.at[pl.ds(2 * (row0 + b * SB), 2 * SB)],
                            ssb1)
            for t in range(2):
                pltpu.sync_copy(dm_hbm.at[t, pl.ds(row0 + b * SB, SB)],
                                dgb1.at[t])
            pltpu.sync_copy(dm64_hbm.at[pl.ds(row0 + b * SB, SB)], dsb1)

    HC = C // 2

    def issue_gather(k, p):
        b = k // SB
        r = k - b * SB
        q = b & 1

        @pl.when(q == 0)
        def _():
            for t in range(2):
                pltpu.async_copy(xl_hbm.at[ssb0.at[2 * r + t]],
                                 xl_b[p].at[pl.ds(t * HC, HC)], gsem[p])
                pltpu.async_copy(xr_hbm.at[dgb0.at[t, r]],
                                 xr_b[p].at[pl.ds(t * HC, HC)], gsem[p])

        @pl.when(q == 1)
        def _():
            for t in range(2):
                pltpu.async_copy(xl_hbm.at[ssb1.at[2 * r + t]],
                                 xl_b[p].at[pl.ds(t * HC, HC)], gsem[p])
                pltpu.async_copy(xr_hbm.at[dgb1.at[t, r]],
                                 xr_b[p].at[pl.ds(t * HC, HC)], gsem[p])

    def wait_gather(p):
        pltpu.make_async_copy(xl_hbm.at[pl.ds(0, C)], xl_b[p], gsem[p]).wait()
        pltpu.make_async_copy(xr_hbm.at[pl.ds(0, C)], xr_b[p], gsem[p]).wait()

    def issue_scatter(k, p):
        b = k // SB
        r = k - b * SB
        q = b & 1

        @pl.when(q == 0)
        def _():
            pltpu.async_copy(xl_b[p], acc_num.at[dsb0.at[r]], ssem[p],
                             add=True)
            pltpu.async_copy(den_b[p], acc_den.at[dsb0.at[r]], ssem[p],
                             add=True)

        @pl.when(q == 1)
        def _():
            pltpu.async_copy(xl_b[p], acc_num.at[dsb1.at[r]], ssem[p],
                             add=True)
            pltpu.async_copy(den_b[p], acc_den.at[dsb1.at[r]], ssem[p],
                             add=True)

    def wait_scatter(p):
        pltpu.make_async_copy(xl_hbm.at[pl.ds(0, C)], xl_b[p], ssem[p]).wait()
        pltpu.make_async_copy(zden_hbm.at[pl.ds(0, C)], den_b[p],
                              ssem[p]).wait()

    def compute(p):
        buf = xl_b[p]
        xr_v = xr_b[p]
        den_v = den_b[p]

        def one_edge(e):
            xl = [buf[e, pl.ds(16 * j, 16)] for j in range(D // 16)]
            t = []
            for j in range(D // 16):
                v = xl[j] + xr_v[e, pl.ds(16 * j, 16)]
                lr = jnp.maximum(v, 0.2 * v)
                t.append(lr * att_r[j])
            svecs = []
            for h in range(H):
                u = bcast_sum(t[2 * h] + t[2 * h + 1])
                svecs.append(jnp.exp(u))
            dval = zero16
            for h in range(H):
                dval = jnp.where(lane_eq[h], svecs[h], dval)
            den_v[e, :] = dval
            for j in range(D // 16):
                buf[e, pl.ds(16 * j, 16)] = xl[j] * svecs[j // 2]

        def edge_body(i, carry):
            one_edge(2 * i)
            one_edge(2 * i + 1)
            return carry

        lax.fori_loop(0, C // 2, edge_body, 0)

    load_sb(0)
    issue_gather(0, 0)

    def pair_body(i, carry):
        k0 = 2 * i
        k1 = k0 + 1

        @pl.when(i > 0)
        def _():
            wait_scatter(1)

        issue_gather(k1, 1)
        wait_gather(0)
        compute(0)
        issue_scatter(k0, 0)

        @pl.when(jnp.logical_and((k1 + 1) % SB == 0, k1 + 1 < K))
        def _():
            load_sb((k1 + 1) // SB)

        wait_gather(1)
        compute(1)

        @pl.when(i > 0)
        def _():
            wait_scatter(0)

        @pl.when(k1 + 1 < K)
        def _():
            issue_gather(k1 + 1, 0)

        issue_scatter(k1, 1)
        return carry

    lax.fori_loop(0, K // 2, pair_body, 0)
    wait_scatter(0)
    wait_scatter(1)
    plsc.subcore_barrier()

    pltpu.sync_copy(acc_num.at[pl.ds(lo, RPT)], onum_hbm.at[c, pl.ds(lo, RPT)])
    pltpu.sync_copy(acc_den.at[pl.ds(lo, RPT)], oden_hbm.at[c, pl.ds(lo, RPT)])


@jax.jit
def _edge_sc(xl_pad, xr_pad, sm, dm, dm64, att):
    znum = jnp.zeros((NP, D), jnp.float32)
    zden = jnp.zeros((NP, 16), jnp.float32)
    mesh = plsc.VectorSubcoreMesh(core_axis_name="c", subcore_axis_name="s")
    f = pl.kernel(
        _edge_body,
        out_type=[
            jax.ShapeDtypeStruct((NC, NP, D), jnp.float32),
            jax.ShapeDtypeStruct((NC, NP, 16), jnp.float32),
        ],
        mesh=mesh,
        scratch_types=[
            pltpu.VMEM_SHARED((NP, D), jnp.float32),
            pltpu.VMEM_SHARED((NP, 16), jnp.float32),
            pltpu.VMEM((C, D), jnp.float32),
            pltpu.VMEM((C, D), jnp.float32),
            pltpu.VMEM((C, D), jnp.float32),
            pltpu.VMEM((C, D), jnp.float32),
            pltpu.VMEM((C, 16), jnp.float32),
            pltpu.VMEM((C, 16), jnp.float32),
            pltpu.VMEM((2 * SB, C // 2), jnp.int32),
            pltpu.VMEM((2 * SB, C // 2), jnp.int32),
            pltpu.VMEM((2, SB, C // 2), jnp.int32),
            pltpu.VMEM((2, SB, C // 2), jnp.int32),
            pltpu.VMEM((SB, C), jnp.int32),
            pltpu.VMEM((SB, C), jnp.int32),
            pltpu.VMEM((D + 32,), jnp.float32),
            pltpu.SemaphoreType.DMA,
            pltpu.SemaphoreType.DMA,
            pltpu.SemaphoreType.DMA,
            pltpu.SemaphoreType.DMA,
        ],
        compiler_params=pltpu.CompilerParams(needs_layout_passes=False,
                                             use_tc_tiling_on_sc=False),
    )
    return f(xl_pad, xr_pad, sm, dm, dm64, att, znum, zden)


def _post_body(num_ref, den_ref, bias_ref, w1_ref, b1_ref, w2_ref, b2_ref,
               g_ref, bt_ref, y_ref):
    num = num_ref[0] + num_ref[1]
    den = den_ref[0, :, :H] + den_ref[1, :, :H]
    den_full = jnp.repeat(den, DH, axis=1)
    h = num / (den_full + 1e-16) + bias_ref[...]
    t = jnp.maximum(jnp.dot(h, w1_ref[...], preferred_element_type=jnp.float32)
                    + b1_ref[...], 0.0)
    y = jnp.dot(t, w2_ref[...], preferred_element_type=jnp.float32) + b2_ref[...] + h
    mean = jnp.mean(y, axis=-1, keepdims=True)
    yc = y - mean
    var = jnp.mean(yc * yc, axis=-1, keepdims=True)
    y_ref[...] = yc * jax.lax.rsqrt(var + 1e-6) * g_ref[...] + bt_ref[...]


@jax.jit
def _post(onum, oden, bias, W1, b1, W2, b2, gamma, beta):
    grid = (N // ROW_BLK,)
    row3 = lambda i: (0, i, 0)
    fixed = lambda i: (0, 0)
    y = pl.pallas_call(
        _post_body,
        grid=grid,
        in_specs=[
            pl.BlockSpec((NC, ROW_BLK, D), row3),
            pl.BlockSpec((NC, ROW_BLK, 16), row3),
            pl.BlockSpec((1, D), fixed),
            pl.BlockSpec((D, D), fixed),
            pl.BlockSpec((1, D), fixed),
            pl.BlockSpec((D, D), fixed),
            pl.BlockSpec((1, D), fixed),
            pl.BlockSpec((1, D), fixed),
            pl.BlockSpec((1, D), fixed),
        ],
        out_specs=pl.BlockSpec((ROW_BLK, D), lambda i: (i, 0)),
        out_shape=jax.ShapeDtypeStruct((N, D), jnp.float32),
    )(onum, oden, bias.reshape(1, D), W1, b1.reshape(1, D), W2,
      b2.reshape(1, D), gamma.reshape(1, D), beta.reshape(1, D))
    return y[None, :, :]


def kernel(x, edge_index, W_l, W_r, att, bias, W1, b1, W2, b2, gamma, beta):
    xl, xr = _proj(x, W_l, W_r)
    pad_rows = jnp.zeros((NP - N, D), jnp.float32)
    xl_pad = jnp.concatenate([xl, pad_rows])
    xr_pad = jnp.concatenate([xr, pad_rows])
    loop = jnp.arange(N, dtype=jnp.int32)
    pad_idx = jnp.full((EN_PAD - EN,), N, jnp.int32)
    src = jnp.concatenate([edge_index[0].astype(jnp.int32), loop, pad_idx])
    dst = jnp.concatenate([edge_index[1].astype(jnp.int32), loop, pad_idx])
    sm = src.reshape(2 * NCHUNK, C // 2)
    dm = dst.reshape(NCHUNK, 2, C // 2).transpose(1, 0, 2)
    dm64 = dst.reshape(NCHUNK, C)
    att_flat = jnp.concatenate([att.reshape(D), jnp.zeros((32,), jnp.float32)])
    onum, oden = _edge_sc(xl_pad, xr_pad, sm, dm, dm64, att_flat)
    return _post(onum, oden, bias, W1, b1, W2, b2, gamma, beta)


# speedup vs baseline: 1.2353x; 1.1523x over previous
import functools

import jax
import jax.numpy as jnp
from jax import lax
from jax.experimental import pallas as pl
from jax.experimental.pallas import tpu as pltpu
from jax.experimental.pallas import tpu_sc as plsc

N = 10000
E = 320000
D = 128
H = 4
DH = D // H

ROW_BLK = 1000

NC = 2
NS = 16
NW = NC * NS
NP = 10112
RPT = NP // NS
EN = E + N
C = 64
K = 168
SB = 8
EN_PAD = NW * K * C
NCHUNK = EN_PAD // C


def _proj_body(x_ref, wl_ref, wr_ref, xl_ref, xr_ref):
    x = x_ref[...]
    xl_ref[...] = jnp.dot(x, wl_ref[...], preferred_element_type=jnp.float32)
    xr_ref[...] = jnp.dot(x, wr_ref[...], preferred_element_type=jnp.float32)


@jax.jit
def _proj(x, W_l, W_r):
    grid = (N // ROW_BLK,)
    return pl.pallas_call(
        _proj_body,
        grid=grid,
        in_specs=[
            pl.BlockSpec((ROW_BLK, D), lambda i: (i, 0)),
            pl.BlockSpec((D, D), lambda i: (0, 0)),
            pl.BlockSpec((D, D), lambda i: (0, 0)),
        ],
        out_specs=[
            pl.BlockSpec((ROW_BLK, D), lambda i: (i, 0)),
            pl.BlockSpec((ROW_BLK, D), lambda i: (i, 0)),
        ],
        out_shape=[
            jax.ShapeDtypeStruct((N, D), jnp.float32),
            jax.ShapeDtypeStruct((N, D), jnp.float32),
        ],
    )(x, W_l, W_r)


def _edge_body(xl_hbm, xr_hbm, sm_hbm, dm_hbm, att_hbm,
               znum_hbm, zden_hbm,
               onum_hbm, oden_hbm,
               acc_num, acc_den,
               xl0, xl1, xr0, xr1, den0, den1,
               ssb0, ssb1, dsb0, dsb1, att_v,
               gsem0, gsem1, ssem0, ssem1):
    c = lax.axis_index("c")
    s = lax.axis_index("s")
    wid = c * NS + s
    lo = pl.multiple_of(s * RPT, 8)
    row0 = wid * K

    xl_b = (xl0, xl1)
    xr_b = (xr0, xr1)
    den_b = (den0, den1)
    gsem = (gsem0, gsem1)
    ssem = (ssem0, ssem1)

    pltpu.sync_copy(znum_hbm.at[pl.ds(lo, RPT)], acc_num.at[pl.ds(lo, RPT)])
    pltpu.sync_copy(zden_hbm.at[pl.ds(lo, RPT)], acc_den.at[pl.ds(lo, RPT)])
    pltpu.sync_copy(att_hbm, att_v)

    zero16 = jnp.zeros((16,), jnp.float32)
    plsc.subcore_barrier()

    lane = lax.iota(jnp.int32, 16)
    xor_idx = [lane ^ 1, lane ^ 2, lane ^ 4, lane ^ 8]
    lane_eq = [lane == h for h in range(H)]
    att_r = [att_v[pl.ds(16 * j, 16)] for j in range(D // 16)]

    def bcast_sum(u):
        dnums = lax.GatherDimensionNumbers(
            offset_dims=(), collapsed_slice_dims=(0,), start_index_map=(0,))
        for xi in xor_idx:
            g = lax.gather(u, xi[:, None], dimension_numbers=dnums,
                           slice_sizes=(1,),
                           mode=lax.GatherScatterMode.PROMISE_IN_BOUNDS)
            u = u + g
        return u

    def load_sb(b):
        q = b & 1

        @pl.when(q == 0)
        def _():
            pltpu.sync_copy(sm_hbm.at[pl.ds(row0 + b * SB, SB)], ssb0)
            pltpu.sync_copy(dm_hbm.at[pl.ds(row0 + b * SB, SB)], dsb0)

        @pl.when(q == 1)
        def _():
            pltpu.sync_copy(sm_hbm.at[pl.ds(row0 + b * SB, SB)], ssb1)
            pltpu.sync_copy(dm_hbm.at[pl.ds(row0 + b * SB, SB)], dsb1)

    def issue_gather(k, p):
        b = k // SB
        r = k - b * SB
        q = b & 1

        @pl.when(q == 0)
        def _():
            pltpu.async_copy(xl_hbm.at[ssb0.at[r]], xl_b[p], gsem[p])
            pltpu.async_copy(xr_hbm.at[dsb0.at[r]], xr_b[p], gsem[p])

        @pl.when(q == 1)
        def _():
            pltpu.async_copy(xl_hbm.at[ssb1.at[r]], xl_b[p], gsem[p])
            pltpu.async_copy(xr_hbm.at[dsb1.at[r]], xr_b[p], gsem[p])

    def wait_gather(p):
        pltpu.make_async_copy(xl_hbm.at[pl.ds(0, C)], xl_b[p], gsem[p]).wait()
        pltpu.make_async_copy(xr_hbm.at[pl.ds(0, C)], xr_b[p], gsem[p]).wait()

    def issue_scatter(k, p):
        b = k // SB
        r = k - b * SB
        q = b & 1

        @pl.when(q == 0)
        def _():
            pltpu.async_copy(xl_b[p], acc_num.at[dsb0.at[r]], ssem[p],
                             add=True)
            pltpu.async_copy(den_b[p], acc_den.at[dsb0.at[r]], ssem[p],
                             add=True)

        @pl.when(q == 1)
        def _():
            pltpu.async_copy(xl_b[p], acc_num.at[dsb1.at[r]], ssem[p],
                             add=True)
            pltpu.async_copy(den_b[p], acc_den.at[dsb1.at[r]], ssem[p],
                             add=True)

    def wait_scatter(p):
        pltpu.make_async_copy(xl_hbm.at[pl.ds(0, C)], xl_b[p], ssem[p]).wait()
        pltpu.make_async_copy(zden_hbm.at[pl.ds(0, C)], den_b[p],
                              ssem[p]).wait()

    def compute(p):
        buf = xl_b[p]
        xr_v = xr_b[p]
        den_v = den_b[p]

        def one_edge(e):
            xl = [buf[e, pl.ds(16 * j, 16)] for j in range(D // 16)]
            t = []
            for j in range(D // 16):
                v = xl[j] + xr_v[e, pl.ds(16 * j, 16)]
                lr = jnp.maximum(v, 0.2 * v)
                t.append(lr * att_r[j])
            svecs = []
            for h in range(H):
                u = bcast_sum(t[2 * h] + t[2 * h + 1])
                svecs.append(jnp.exp(u))
            dval = zero16
            for h in range(H):
                dval = jnp.where(lane_eq[h], svecs[h], dval)
            den_v[e, :] = dval
            for j in range(D // 16):
                buf[e, pl.ds(16 * j, 16)] = xl[j] * svecs[j // 2]

        def edge_body(i, carry):
            one_edge(2 * i)
            one_edge(2 * i + 1)
            return carry

        lax.fori_loop(0, C // 2, edge_body, 0)

    load_sb(0)
    issue_gather(0, 0)

    def pair_body(i, carry):
        k0 = 2 * i
        k1 = k0 + 1

        @pl.when(i > 0)
        def _():
            wait_scatter(1)

        issue_gather(k1, 1)
        wait_gather(0)
        compute(0)
        issue_scatter(k0, 0)

        @pl.when(jnp.logical_and((k1 + 1) % SB == 0, k1 + 1 < K))
        def _():
            load_sb((k1 + 1) // SB)

        wait_gather(1)
        compute(1)

        @pl.when(i > 0)
        def _():
            wait_scatter(0)

        @pl.when(k1 + 1 < K)
        def _():
            issue_gather(k1 + 1, 0)

        issue_scatter(k1, 1)
        return carry

    lax.fori_loop(0, K // 2, pair_body, 0)
    wait_scatter(0)
    wait_scatter(1)
    plsc.subcore_barrier()

    pltpu.sync_copy(acc_num.at[pl.ds(lo, RPT)], onum_hbm.at[c, pl.ds(lo, RPT)])
    pltpu.sync_copy(acc_den.at[pl.ds(lo, RPT)], oden_hbm.at[c, pl.ds(lo, RPT)])


@jax.jit
def _edge_sc(xl_pad, xr_pad, sm, dm, att):
    znum = jnp.zeros((NP, D), jnp.float32)
    zden = jnp.zeros((NP, 16), jnp.float32)
    mesh = plsc.VectorSubcoreMesh(core_axis_name="c", subcore_axis_name="s")
    f = pl.kernel(
        _edge_body,
        out_type=[
            jax.ShapeDtypeStruct((NC, NP, D), jnp.float32),
            jax.ShapeDtypeStruct((NC, NP, 16), jnp.float32),
        ],
        mesh=mesh,
        scratch_types=[
            pltpu.VMEM_SHARED((NP, D), jnp.float32),
            pltpu.VMEM_SHARED((NP, 16), jnp.float32),
            pltpu.VMEM((C, D), jnp.float32),
            pltpu.VMEM((C, D), jnp.float32),
            pltpu.VMEM((C, D), jnp.float32),
            pltpu.VMEM((C, D), jnp.float32),
            pltpu.VMEM((C, 16), jnp.float32),
            pltpu.VMEM((C, 16), jnp.float32),
            pltpu.VMEM((SB, C), jnp.int32),
            pltpu.VMEM((SB, C), jnp.int32),
            pltpu.VMEM((SB, C), jnp.int32),
            pltpu.VMEM((SB, C), jnp.int32),
            pltpu.VMEM((D + 32,), jnp.float32),
            pltpu.SemaphoreType.DMA,
            pltpu.SemaphoreType.DMA,
            pltpu.SemaphoreType.DMA,
            pltpu.SemaphoreType.DMA,
        ],
        compiler_params=pltpu.CompilerParams(needs_layout_passes=False,
                                             use_tc_tiling_on_sc=False),
    )
    return f(xl_pad, xr_pad, sm, dm, att, znum, zden)


def _post_body(num_ref, den_ref, bias_ref, w1_ref, b1_ref, w2_ref, b2_ref,
               g_ref, bt_ref, y_ref):
    num = num_ref[0] + num_ref[1]
    den = den_ref[0, :, :H] + den_ref[1, :, :H]
    den_full = jnp.repeat(den, DH, axis=1)
    h = num / (den_full + 1e-16) + bias_ref[...]
    t = jnp.maximum(jnp.dot(h, w1_ref[...], preferred_element_type=jnp.float32)
                    + b1_ref[...], 0.0)
    y = jnp.dot(t, w2_ref[...], preferred_element_type=jnp.float32) + b2_ref[...] + h
    mean = jnp.mean(y, axis=-1, keepdims=True)
    yc = y - mean
    var = jnp.mean(yc * yc, axis=-1, keepdims=True)
    y_ref[...] = yc * jax.lax.rsqrt(var + 1e-6) * g_ref[...] + bt_ref[...]


@jax.jit
def _post(onum, oden, bias, W1, b1, W2, b2, gamma, beta):
    grid = (N // ROW_BLK,)
    row3 = lambda i: (0, i, 0)
    fixed = lambda i: (0, 0)
    y = pl.pallas_call(
        _post_body,
        grid=grid,
        in_specs=[
            pl.BlockSpec((NC, ROW_BLK, D), row3),
            pl.BlockSpec((NC, ROW_BLK, 16), row3),
            pl.BlockSpec((1, D), fixed),
            pl.BlockSpec((D, D), fixed),
            pl.BlockSpec((1, D), fixed),
            pl.BlockSpec((D, D), fixed),
            pl.BlockSpec((1, D), fixed),
            pl.BlockSpec((1, D), fixed),
            pl.BlockSpec((1, D), fixed),
        ],
        out_specs=pl.BlockSpec((ROW_BLK, D), lambda i: (i, 0)),
        out_shape=jax.ShapeDtypeStruct((N, D), jnp.float32),
    )(onum, oden, bias.reshape(1, D), W1, b1.reshape(1, D), W2,
      b2.reshape(1, D), gamma.reshape(1, D), beta.reshape(1, D))
    return y[None, :, :]


def kernel(x, edge_index, W_l, W_r, att, bias, W1, b1, W2, b2, gamma, beta):
    xl, xr = _proj(x, W_l, W_r)
    pad_rows = jnp.zeros((NP - N, D), jnp.float32)
    xl_pad = jnp.concatenate([xl, pad_rows])
    xr_pad = jnp.concatenate([xr, pad_rows])
    loop = jnp.arange(N, dtype=jnp.int32)
    pad_idx = jnp.full((EN_PAD - EN,), N, jnp.int32)
    src = jnp.concatenate([edge_index[0].astype(jnp.int32), loop, pad_idx])
    dst = jnp.concatenate([edge_index[1].astype(jnp.int32), loop, pad_idx])
    sm = src.reshape(NCHUNK, C)
    dm = dst.reshape(NCHUNK, C)
    att_flat = jnp.concatenate([att.reshape(D), jnp.zeros((32,), jnp.float32)])
    onum, oden = _edge_sc(xl_pad, xr_pad, sm, dm, att_flat)
    return _post(onum, oden, bias, W1, b1, W2, b2, gamma, beta)


# speedup vs baseline: 1.8408x; 1.4902x over previous
import functools

import jax
import jax.numpy as jnp
from jax import lax
from jax.experimental import pallas as pl
from jax.experimental.pallas import tpu as pltpu
from jax.experimental.pallas import tpu_sc as plsc

N = 10000
E = 320000
D = 128
H = 4
DH = D // H

ROW_BLK = 1000

NC = 2
NS = 16
NW = NC * NS
NP = 10112
RPT = NP // NS
C = 64
K = 158
SB = 8
EN_PAD = NW * K * C
NCHUNK = EN_PAD // C
IDX_ALLOC = 5064


def _proj_body(x_ref, wl_ref, wr_ref, xl_ref, xr_ref):
    x = x_ref[...]
    xl_ref[...] = jnp.dot(x, wl_ref[...], preferred_element_type=jnp.float32)
    xr_ref[...] = jnp.dot(x, wr_ref[...], preferred_element_type=jnp.float32)


@jax.jit
def _proj(x, W_l, W_r):
    grid = (N // ROW_BLK,)
    return pl.pallas_call(
        _proj_body,
        grid=grid,
        in_specs=[
            pl.BlockSpec((ROW_BLK, D), lambda i: (i, 0)),
            pl.BlockSpec((D, D), lambda i: (0, 0)),
            pl.BlockSpec((D, D), lambda i: (0, 0)),
        ],
        out_specs=[
            pl.BlockSpec((ROW_BLK, D), lambda i: (i, 0)),
            pl.BlockSpec((ROW_BLK, D), lambda i: (i, 0)),
        ],
        out_shape=[
            jax.ShapeDtypeStruct((N, D), jnp.float32),
            jax.ShapeDtypeStruct((N, D), jnp.float32),
        ],
    )(x, W_l, W_r)


def _edge_body(xl_hbm, xr_hbm, sm_hbm, dm_hbm, att_hbm,
               znum_hbm, zden_hbm,
               onum_hbm, oden_hbm,
               acc_num, acc_den,
               xl0, xl1, xr0, xr1, den0, den1,
               ssb0, ssb1, dsb0, dsb1, att_v,
               gsem0, gsem1, ssem0, ssem1):
    c = lax.axis_index("c")
    s = lax.axis_index("s")
    wid = c * NS + s
    lo = pl.multiple_of(s * RPT, 8)
    row0 = wid * K

    xl_b = (xl0, xl1)
    xr_b = (xr0, xr1)
    den_b = (den0, den1)
    gsem = (gsem0, gsem1)
    ssem = (ssem0, ssem1)

    pltpu.sync_copy(znum_hbm.at[pl.ds(lo, RPT)], acc_num.at[pl.ds(lo, RPT)])
    pltpu.sync_copy(zden_hbm.at[pl.ds(lo, RPT)], acc_den.at[pl.ds(lo, RPT)])
    pltpu.sync_copy(att_hbm, att_v)

    zero16 = jnp.zeros((16,), jnp.float32)
    plsc.subcore_barrier()

    lane = lax.iota(jnp.int32, 16)
    xor_idx = [lane ^ 1, lane ^ 2, lane ^ 4, lane ^ 8]
    lane_eq = [lane == h for h in range(H)]
    att_r = [att_v[pl.ds(16 * j, 16)] for j in range(D // 16)]

    def bcast_sum(u):
        dnums = lax.GatherDimensionNumbers(
            offset_dims=(), collapsed_slice_dims=(0,), start_index_map=(0,))
        for xi in xor_idx:
            g = lax.gather(u, xi[:, None], dimension_numbers=dnums,
                           slice_sizes=(1,),
                           mode=lax.GatherScatterMode.PROMISE_IN_BOUNDS)
            u = u + g
        return u

    def load_sb(b):
        q = b & 1

        @pl.when(q == 0)
        def _():
            pltpu.sync_copy(sm_hbm.at[pl.ds(row0 + b * SB, SB)], ssb0)
            pltpu.sync_copy(dm_hbm.at[pl.ds(row0 + b * SB, SB)], dsb0)

        @pl.when(q == 1)
        def _():
            pltpu.sync_copy(sm_hbm.at[pl.ds(row0 + b * SB, SB)], ssb1)
            pltpu.sync_copy(dm_hbm.at[pl.ds(row0 + b * SB, SB)], dsb1)

    def issue_gather(k, p):
        b = k // SB
        r = k - b * SB
        q = b & 1

        @pl.when(q == 0)
        def _():
            pltpu.async_copy(xl_hbm.at[ssb0.at[r]], xl_b[p], gsem[p])
            pltpu.async_copy(xr_hbm.at[dsb0.at[r]], xr_b[p], gsem[p])

        @pl.when(q == 1)
        def _():
            pltpu.async_copy(xl_hbm.at[ssb1.at[r]], xl_b[p], gsem[p])
            pltpu.async_copy(xr_hbm.at[dsb1.at[r]], xr_b[p], gsem[p])

    def wait_gather(p):
        pltpu.make_async_copy(xl_hbm.at[pl.ds(0, C)], xl_b[p], gsem[p]).wait()
        pltpu.make_async_copy(xr_hbm.at[pl.ds(0, C)], xr_b[p], gsem[p]).wait()

    def issue_scatter(k, p):
        b = k // SB
        r = k - b * SB
        q = b & 1

        @pl.when(q == 0)
        def _():
            pltpu.async_copy(xl_b[p], acc_num.at[dsb0.at[r]], ssem[p],
                             add=True)
            pltpu.async_copy(den_b[p], acc_den.at[dsb0.at[r]], ssem[p],
                             add=True)

        @pl.when(q == 1)
        def _():
            pltpu.async_copy(xl_b[p], acc_num.at[dsb1.at[r]], ssem[p],
                             add=True)
            pltpu.async_copy(den_b[p], acc_den.at[dsb1.at[r]], ssem[p],
                             add=True)

    def wait_scatter(p):
        pltpu.make_async_copy(xl_hbm.at[pl.ds(0, C)], xl_b[p], ssem[p]).wait()
        pltpu.make_async_copy(zden_hbm.at[pl.ds(0, C)], den_b[p],
                              ssem[p]).wait()

    def compute(p):
        buf = xl_b[p]
        xr_v = xr_b[p]
        den_v = den_b[p]

        def one_edge(e):
            xl = [buf[e, pl.ds(16 * j, 16)] for j in range(D // 16)]
            t = []
            for j in range(D // 16):
                v = xl[j] + xr_v[e, pl.ds(16 * j, 16)]
                lr = jnp.maximum(v, 0.2 * v)
                t.append(lr * att_r[j])
            svecs = []
            for h in range(H):
                u = bcast_sum(t[2 * h] + t[2 * h + 1])
                svecs.append(jnp.exp(u))
            dval = zero16
            for h in range(H):
                dval = jnp.where(lane_eq[h], svecs[h], dval)
            den_v[e, :] = dval
            for j in range(D // 16):
                buf[e, pl.ds(16 * j, 16)] = xl[j] * svecs[j // 2]

        def edge_body(i, carry):
            one_edge(2 * i)
            one_edge(2 * i + 1)
            return carry

        lax.fori_loop(0, C // 2, edge_body, 0)

    load_sb(0)
    issue_gather(0, 0)

    def pair_body(i, carry):
        k0 = 2 * i
        k1 = k0 + 1

        @pl.when(i > 0)
        def _():
            wait_scatter(1)

        issue_gather(k1, 1)
        wait_gather(0)
        compute(0)
        issue_scatter(k0, 0)

        @pl.when(jnp.logical_and((k1 + 1) % SB == 0, k1 + 1 < K))
        def _():
            load_sb((k1 + 1) // SB)

        wait_gather(1)
        compute(1)

        @pl.when(i > 0)
        def _():
            wait_scatter(0)

        @pl.when(k1 + 1 < K)
        def _():
            issue_gather(k1 + 1, 0)

        issue_scatter(k1, 1)
        return carry

    lax.fori_loop(0, K // 2, pair_body, 0)
    wait_scatter(0)
    wait_scatter(1)
    plsc.subcore_barrier()

    pltpu.sync_copy(acc_num.at[pl.ds(lo, RPT)], onum_hbm.at[c, pl.ds(lo, RPT)])
    pltpu.sync_copy(acc_den.at[pl.ds(lo, RPT)], oden_hbm.at[c, pl.ds(lo, RPT)])


@jax.jit
def _edge_sc(xl_pad, xr_pad, sm, dm, att):
    znum = jnp.zeros((NP, D), jnp.float32)
    zden = jnp.zeros((NP, 16), jnp.float32)
    mesh = plsc.VectorSubcoreMesh(core_axis_name="c", subcore_axis_name="s")
    f = pl.kernel(
        _edge_body,
        out_type=[
            jax.ShapeDtypeStruct((NC, NP, D), jnp.float32),
            jax.ShapeDtypeStruct((NC, NP, 16), jnp.float32),
        ],
        mesh=mesh,
        scratch_types=[
            pltpu.VMEM_SHARED((NP, D), jnp.float32),
            pltpu.VMEM_SHARED((NP, 16), jnp.float32),
            pltpu.VMEM((C, D), jnp.float32),
            pltpu.VMEM((C, D), jnp.float32),
            pltpu.VMEM((C, D), jnp.float32),
            pltpu.VMEM((C, D), jnp.float32),
            pltpu.VMEM((C, 16), jnp.float32),
            pltpu.VMEM((C, 16), jnp.float32),
            pltpu.VMEM((SB, C), jnp.int32),
            pltpu.VMEM((SB, C), jnp.int32),
            pltpu.VMEM((SB, C), jnp.int32),
            pltpu.VMEM((SB, C), jnp.int32),
            pltpu.VMEM((D + 32,), jnp.float32),
            pltpu.SemaphoreType.DMA,
            pltpu.SemaphoreType.DMA,
            pltpu.SemaphoreType.DMA,
            pltpu.SemaphoreType.DMA,
        ],
        compiler_params=pltpu.CompilerParams(needs_layout_passes=False,
                                             use_tc_tiling_on_sc=False),
    )
    return f(xl_pad, xr_pad, sm, dm, att, znum, zden)


def _post_body(num_ref, den_ref, xl_ref, xr_ref, att_ref, bias_ref,
               w1_ref, b1_ref, w2_ref, b2_ref, g_ref, bt_ref, y_ref):
    xl = xl_ref[...]
    e = xl + xr_ref[...]
    lr = jnp.where(e > 0, e, 0.2 * e)
    prod = lr * att_ref[...]
    r = lax.broadcasted_iota(jnp.int32, (D, H), 0)
    col = lax.broadcasted_iota(jnp.int32, (D, H), 1)
    M = (r // DH == col).astype(jnp.float32)
    s_self = jnp.exp(jnp.dot(prod, M, preferred_element_type=jnp.float32))
    num = num_ref[0] + num_ref[1] + jnp.repeat(s_self, DH, axis=1) * xl
    den = den_ref[0, :, :H] + den_ref[1, :, :H] + s_self
    den_full = jnp.repeat(den, DH, axis=1)
    h = num / (den_full + 1e-16) + bias_ref[...]
    t = jnp.maximum(jnp.dot(h, w1_ref[...], preferred_element_type=jnp.float32)
                    + b1_ref[...], 0.0)
    y = jnp.dot(t, w2_ref[...], preferred_element_type=jnp.float32) + b2_ref[...] + h
    mean = jnp.mean(y, axis=-1, keepdims=True)
    yc = y - mean
    var = jnp.mean(yc * yc, axis=-1, keepdims=True)
    y_ref[...] = yc * jax.lax.rsqrt(var + 1e-6) * g_ref[...] + bt_ref[...]


@jax.jit
def _post(onum, oden, xl, xr, att, bias, W1, b1, W2, b2, gamma, beta):
    grid = (N // ROW_BLK,)
    row3 = lambda i: (0, i, 0)
    row = lambda i: (i, 0)
    fixed = lambda i: (0, 0)
    y = pl.pallas_call(
        _post_body,
        grid=grid,
        in_specs=[
            pl.BlockSpec((NC, ROW_BLK, D), row3),
            pl.BlockSpec((NC, ROW_BLK, 16), row3),
            pl.BlockSpec((ROW_BLK, D), row),
            pl.BlockSpec((ROW_BLK, D), row),
            pl.BlockSpec((1, D), fixed),
            pl.BlockSpec((1, D), fixed),
            pl.BlockSpec((D, D), fixed),
            pl.BlockSpec((1, D), fixed),
            pl.BlockSpec((D, D), fixed),
            pl.BlockSpec((1, D), fixed),
            pl.BlockSpec((1, D), fixed),
            pl.BlockSpec((1, D), fixed),
        ],
        out_specs=pl.BlockSpec((ROW_BLK, D), lambda i: (i, 0)),
        out_shape=jax.ShapeDtypeStruct((N, D), jnp.float32),
    )(onum, oden, xl, xr, att.reshape(1, D), bias.reshape(1, D), W1,
      b1.reshape(1, D), W2, b2.reshape(1, D), gamma.reshape(1, D),
      beta.reshape(1, D))
    return y[None, :, :]


def kernel(x, edge_index, W_l, W_r, att, bias, W1, b1, W2, b2, gamma, beta):
    xl, xr = _proj(x, W_l, W_r)
    pad_rows = jnp.zeros((NP - N, D), jnp.float32)
    xl_pad = jnp.concatenate([xl, pad_rows])
    xr_pad = jnp.concatenate([xr, pad_rows])
    pad_idx = jnp.full((IDX_ALLOC * C - E,), N, jnp.int32)
    src = jnp.concatenate([edge_index[0].astype(jnp.int32), pad_idx])
    dst = jnp.concatenate([edge_index[1].astype(jnp.int32), pad_idx])
    sm = src.reshape(IDX_ALLOC, C)
    dm = dst.reshape(IDX_ALLOC, C)
    att_flat = jnp.concatenate([att.reshape(D), jnp.zeros((32,), jnp.float32)])
    onum, oden = _edge_sc(xl_pad, xr_pad, sm, dm, att_flat)
    return _post(onum, oden, xl, xr, att.reshape(1, D), bias, W1, b1, W2, b2,
                 gamma, beta)
